# baseline (speedup 1.0000x reference)
import functools

import jax
import jax.numpy as jnp
from jax import lax
from jax.experimental import pallas as pl
from jax.experimental.pallas import tpu as pltpu
from jax.experimental.pallas import tpu_sc as plsc

N = 10000
D = 256
E = 160000
DH = 128
C = 80
L = 16
NSUB = 16
EPT = E // NSUB
CPT = EPT // C
DEG_SPLIT = (CPT + 1) // 2
SUPER = 25
SUPER_E = SUPER * C
FLUSH_ROWS = (N // NSUB) // 8 * 8
ALPHA = 0.5
EPS = 1e-5


def _sc_aggregate(x_cat, src, dst):
    mesh = plsc.VectorSubcoreMesh(core_axis_name="core",
                                  subcore_axis_name="subcore")
    out_type = [
        jax.ShapeDtypeStruct((2 * N, DH), jnp.float32),
        jax.ShapeDtypeStruct((2 * N,), jnp.float32),
    ]
    scratch = [
        pltpu.VMEM_SHARED((N, DH), jnp.float32),
        pltpu.VMEM_SHARED((N,), jnp.float32),
        pltpu.VMEM((SUPER_E,), jnp.int32),
        pltpu.VMEM((SUPER_E,), jnp.int32),
        pltpu.VMEM((C,), jnp.int32),
        pltpu.VMEM((C,), jnp.int32),
        pltpu.VMEM((C, DH), jnp.float32),
        pltpu.VMEM((C,), jnp.int32),
        pltpu.VMEM((C,), jnp.int32),
        pltpu.VMEM((C, DH), jnp.float32),
        pltpu.VMEM((C,), jnp.int32),
        pltpu.VMEM((C,), jnp.int32),
        pltpu.VMEM((C, DH), jnp.float32),
        pltpu.VMEM((C,), jnp.float32),
        pltpu.VMEM((C,), jnp.float32),
        pltpu.VMEM((FLUSH_ROWS + 16,), jnp.float32),
        pltpu.SemaphoreType.DMA,
        pltpu.SemaphoreType.DMA,
        pltpu.SemaphoreType.DMA,
        pltpu.SemaphoreType.DMA,
        pltpu.SemaphoreType.DMA,
        pltpu.SemaphoreType.DMA,
        pltpu.SemaphoreType.DMA,
        pltpu.SemaphoreType.DMA,
        pltpu.SemaphoreType.DMA,
    ]

    @functools.partial(pl.kernel, mesh=mesh, out_type=out_type,
                       scratch_types=scratch)
    def k(x_hbm, s_hbm, d_hbm, agg_hbm, deg_hbm,
          sh_agg, sh_deg, srcbig, dstbig,
          idx0, dst0, rows0, idx1, dst1, rows1, idx2, dst2, rows2,
          ones_v, zdeg_v, dbuf,
          gsem0, gsem1, gsem2, ssem0, ssem1, ssem2, dsem0, dsem1, dsem2):
        c = lax.axis_index("core")
        s = lax.axis_index("subcore")
        coff = c * N
        ebase = s * EPT

        for j in range(C // L):
            ones_v[pl.ds(j * L, L)] = jnp.full((L,), 1.0, jnp.float32)
            zdeg_v[pl.ds(j * L, L)] = jnp.zeros((L,), jnp.float32)

        @pl.loop(0, C)
        def _(i):
            for j in range(DH // L):
                rows0[i, pl.ds(j * L, L)] = jnp.zeros((L,), jnp.float32)

        fb = s * FLUSH_ROWS
        zlens = [C] * (FLUSH_ROWS // C) + [FLUSH_ROWS % C]
        off = 0
        for ln in zlens:
            pltpu.sync_copy(rows0.at[pl.ds(0, ln)],
                            sh_agg.at[pl.ds(fb + off, ln)])
            pltpu.sync_copy(zdeg_v.at[pl.ds(0, ln)],
                            sh_deg.at[pl.ds(fb + off, ln)])
            off += ln

        @pl.when(s == NSUB - 1)
        def _():
            tb = NSUB * FLUSH_ROWS
            pltpu.sync_copy(rows0.at[pl.ds(0, N - tb)],
                            sh_agg.at[pl.ds(tb, N - tb)])
            pltpu.sync_copy(zdeg_v.at[pl.ds(0, N - tb)],
                            sh_deg.at[pl.ds(tb, N - tb)])
        plsc.subcore_barrier()

        def deg_on(j):
            return ((j < DEG_SPLIT) & (c == 0)) | ((j >= DEG_SPLIT) & (c == 1))

        def superload(g):
            e0 = ebase + g * SUPER_E
            pltpu.sync_copy(s_hbm.at[pl.ds(e0, SUPER_E)], srcbig)
            pltpu.sync_copy(d_hbm.at[pl.ds(e0, SUPER_E)], dstbig)

        def prep(j, ib, db):
            off_ = (j % SUPER) * C
            for r in range(C // L):
                sl = pl.ds(off_ + r * L, L)
                ib[pl.ds(r * L, L)] = srcbig[sl] + coff
                db[pl.ds(r * L, L)] = dstbig[sl]

        def scatter_start(j, rb, db, ss, ds_):
            pltpu.async_copy(rb, sh_agg.at[db], ss, add=True)

            @pl.when(deg_on(j))
            def _():
                pltpu.async_copy(ones_v, sh_deg.at[db], ds_, add=True)

        def scatter_wait(j, rb, db, ss, ds_):
            pltpu.make_async_copy(rb, sh_agg.at[db], ss).wait()

            @pl.when(deg_on(j))
            def _():
                pltpu.make_async_copy(ones_v, sh_deg.at[db], ds_).wait()

        def gather_start(ib, rb, gs):
            pltpu.async_copy(x_hbm.at[ib], rb, gs)

        def gather_wait(ib, rb, gs):
            pltpu.make_async_copy(x_hbm.at[ib], rb, gs).wait()

        def halfstep(j, P, PN):
            ib, db, rb, gs, ss, ds_ = P
            ibn, dbn, rbn, gsn, ssn, dsn = PN

            @pl.when(j >= 2)
            def _():
                scatter_wait(j - 2, rbn, dbn, ssn, dsn)

            @pl.when((j + 1) % SUPER == 0)
            def _():
                superload((j + 1) // SUPER)

            prep(j + 1, ibn, dbn)
            gather_wait(ib, rb, gs)
            gather_start(ibn, rbn, gsn)
            scatter_start(j, rb, db, ss, ds_)

        B0 = (idx0, dst0, rows0, gsem0, ssem0, dsem0)
        B1 = (idx1, dst1, rows1, gsem1, ssem1, dsem1)
        B2 = (idx2, dst2, rows2, gsem2, ssem2, dsem2)

        superload(0)
        prep(0, idx0, dst0)
        gather_start(idx0, rows0, gsem0)

        @pl.loop(0, (CPT - 2) // 3)
        def _(t):
            halfstep(3 * t, B0, B1)
            halfstep(3 * t + 1, B1, B2)
            halfstep(3 * t + 2, B2, B0)

        scatter_wait(CPT - 4, rows1, dst1, ssem1, dsem1)
        prep(CPT - 1, idx1, dst1)
        gather_wait(idx0, rows0, gsem0)
        gather_start(idx1, rows1, gsem1)
        scatter_start(CPT - 2, rows0, dst0, ssem0, dsem0)

        scatter_wait(CPT - 3, rows2, dst2, ssem2, dsem2)
        gather_wait(idx1, rows1, gsem1)
        scatter_start(CPT - 1, rows1, dst1, ssem1, dsem1)

        scatter_wait(CPT - 2, rows0, dst0, ssem0, dsem0)
        scatter_wait(CPT - 1, rows1, dst1, ssem1, dsem1)

        plsc.subcore_barrier()

        off = 0
        for ln in zlens:
            pltpu.sync_copy(sh_agg.at[pl.ds(fb + off, ln)],
                            rows0.at[pl.ds(0, ln)])
            pltpu.sync_copy(rows0.at[pl.ds(0, ln)],
                            agg_hbm.at[pl.ds(coff + fb + off, ln)])
            off += ln
        pltpu.sync_copy(sh_deg.at[pl.ds(fb, FLUSH_ROWS)],
                        dbuf.at[pl.ds(0, FLUSH_ROWS)])
        pltpu.sync_copy(dbuf.at[pl.ds(0, FLUSH_ROWS)],
                        deg_hbm.at[pl.ds(coff + fb, FLUSH_ROWS)])

        @pl.when(s == NSUB - 1)
        def _():
            tb = NSUB * FLUSH_ROWS
            pltpu.sync_copy(sh_agg.at[pl.ds(tb, N - tb)],
                            rows0.at[pl.ds(0, N - tb)])
            pltpu.sync_copy(rows0.at[pl.ds(0, N - tb)],
                            agg_hbm.at[pl.ds(coff + tb, N - tb)])
            pltpu.sync_copy(sh_deg.at[pl.ds(tb, N - tb)],
                            dbuf.at[pl.ds(0, N - tb)])
            pltpu.sync_copy(dbuf.at[pl.ds(0, N - tb)],
                            deg_hbm.at[pl.ds(coff + tb, N - tb)])

    return k(x_cat, src, dst)


def _norm_body(xu, aul, auh, dul, duh, xi, ail, aih, dil, dih,
               pu, pi, gu, bu, gi, bi, ou, oi):
    for x, alo, ahi, dlo, dhi, p, g, b, o in (
            (xu, aul, auh, dul, duh, pu, gu, bu, ou),
            (xi, ail, aih, dil, dih, pi, gi, bi, oi)):
        a = jnp.concatenate([alo[...], ahi[...]], axis=1)
        d = jnp.maximum(dlo[...] + dhi[...], 1.0)
        h = x[...] + (ALPHA * (a * p[...])) / d
        mu = jnp.mean(h, axis=-1, keepdims=True)
        var = jnp.mean((h - mu) ** 2, axis=-1, keepdims=True)
        o[...] = (h - mu) * lax.rsqrt(var + EPS) * g[...] + b[...]


def _tc_normalize(x_user, aggu_cat, degu_cat, x_item, aggi_cat, degi_cat,
                  p_iu, p_ui, g_u, b_u, g_i, b_i):
    R = 1000
    bs_feat = pl.BlockSpec((R, D), lambda i: (i, 0))
    bs_lo = pl.BlockSpec((R, DH), lambda i: (i, 0))
    bs_hi = pl.BlockSpec((R, DH), lambda i: (N // R + i, 0))
    bs_dlo = pl.BlockSpec((R, 1), lambda i: (i, 0))
    bs_dhi = pl.BlockSpec((R, 1), lambda i: (N // R + i, 0))
    bs_vec = pl.BlockSpec((1, D), lambda i: (0, 0))
    return pl.pallas_call(
        _norm_body,
        grid=(N // R,),
        in_specs=[bs_feat, bs_lo, bs_hi, bs_dlo, bs_dhi,
                  bs_feat, bs_lo, bs_hi, bs_dlo, bs_dhi,
                  bs_vec, bs_vec, bs_vec, bs_vec, bs_vec, bs_vec],
        out_specs=[bs_feat, bs_feat],
        out_shape=[jax.ShapeDtypeStruct((N, D), jnp.float32),
                   jax.ShapeDtypeStruct((N, D), jnp.float32)],
    )(x_user, aggu_cat, aggu_cat, degu_cat, degu_cat,
      x_item, aggi_cat, aggi_cat, degi_cat, degi_cat,
      p_iu.reshape(1, D), p_ui.reshape(1, D),
      g_u.reshape(1, D), b_u.reshape(1, D),
      g_i.reshape(1, D), b_i.reshape(1, D))


def kernel(x_user, x_item, edge_user_item, edge_item_user,
           p_user_item, p_item_user, g_user, b_user, g_item, b_item):
    xu_cat = jnp.concatenate([x_user[:, :DH], x_user[:, DH:]], axis=0)
    xi_cat = jnp.concatenate([x_item[:, :DH], x_item[:, DH:]], axis=0)
    aggi_cat, degi_cat = _sc_aggregate(
        xu_cat, edge_user_item[0], edge_user_item[1])
    aggu_cat, degu_cat = _sc_aggregate(
        xi_cat, edge_item_user[0], edge_item_user[1])
    out_user, out_item = _tc_normalize(
        x_user, aggu_cat, degu_cat.reshape(2 * N, 1),
        x_item, aggi_cat, degi_cat.reshape(2 * N, 1),
        p_item_user, p_user_item, g_user, b_user, g_item, b_item)
    return (out_user, out_item)


# speedup vs baseline: 1.0283x; 1.0283x over previous
import functools

import jax
import jax.numpy as jnp
from jax import lax
from jax.experimental import pallas as pl
from jax.experimental.pallas import tpu as pltpu
from jax.experimental.pallas import tpu_sc as plsc

N = 10000
D = 256
E = 160000
DH = 128
C = 80
L = 16
NSUB = 16
EPT = E // NSUB
CPT = EPT // C
DEG_SPLIT = (CPT + 1) // 2
SUPER = 25
SUPER_E = SUPER * C
FLUSH_ROWS = (N // NSUB) // 8 * 8
ALPHA = 0.5
EPS = 1e-5


def _sc_aggregate(x_cat, src, dst):
    mesh = plsc.VectorSubcoreMesh(core_axis_name="core",
                                  subcore_axis_name="subcore")
    out_type = [
        jax.ShapeDtypeStruct((2 * N, DH), jnp.float32),
        jax.ShapeDtypeStruct((2 * N,), jnp.float32),
    ]
    scratch = [
        pltpu.VMEM_SHARED((N, DH), jnp.float32),
        pltpu.VMEM_SHARED((N,), jnp.float32),
        pltpu.VMEM((SUPER_E,), jnp.int32),
        pltpu.VMEM((SUPER_E,), jnp.int32),
        pltpu.VMEM((C,), jnp.int32),
        pltpu.VMEM((C,), jnp.int32),
        pltpu.VMEM((C, DH), jnp.float32),
        pltpu.VMEM((C,), jnp.int32),
        pltpu.VMEM((C,), jnp.int32),
        pltpu.VMEM((C, DH), jnp.float32),
        pltpu.VMEM((C,), jnp.int32),
        pltpu.VMEM((C,), jnp.int32),
        pltpu.VMEM((C, DH), jnp.float32),
        pltpu.VMEM((C,), jnp.float32),
        pltpu.VMEM((C,), jnp.float32),
        pltpu.VMEM((FLUSH_ROWS + 16,), jnp.float32),
        pltpu.SemaphoreType.DMA,
        pltpu.SemaphoreType.DMA,
        pltpu.SemaphoreType.DMA,
        pltpu.SemaphoreType.DMA,
        pltpu.SemaphoreType.DMA,
        pltpu.SemaphoreType.DMA,
        pltpu.SemaphoreType.DMA,
        pltpu.SemaphoreType.DMA,
        pltpu.SemaphoreType.DMA,
    ]

    @functools.partial(pl.kernel, mesh=mesh, out_type=out_type,
                       scratch_types=scratch)
    def k(x_hbm, s_hbm, d_hbm, agg_hbm, deg_hbm,
          sh_agg, sh_deg, srcbig, dstbig,
          idx0, dst0, rows0, idx1, dst1, rows1, idx2, dst2, rows2,
          ones_v, zdeg_v, dbuf,
          gsem0, gsem1, gsem2, ssem0, ssem1, ssem2, dsem0, dsem1, dsem2):
        c = lax.axis_index("core")
        s = lax.axis_index("subcore")
        coff = c * N
        ebase = s * EPT

        for j in range(C // L):
            ones_v[pl.ds(j * L, L)] = jnp.full((L,), 1.0, jnp.float32)
            zdeg_v[pl.ds(j * L, L)] = jnp.zeros((L,), jnp.float32)

        fb = s * FLUSH_ROWS
        zlens = [C] * (FLUSH_ROWS // C) + [FLUSH_ROWS % C]

        def deg_on(j):
            return ((j < DEG_SPLIT) & (c == 0)) | ((j >= DEG_SPLIT) & (c == 1))

        def superload(g):
            e0 = ebase + g * SUPER_E
            pltpu.sync_copy(s_hbm.at[pl.ds(e0, SUPER_E)], srcbig)
            pltpu.sync_copy(d_hbm.at[pl.ds(e0, SUPER_E)], dstbig)

        def prep(j, ib, db):
            off_ = (j % SUPER) * C
            for r in range(C // L):
                sl = pl.ds(off_ + r * L, L)
                ib[pl.ds(r * L, L)] = srcbig[sl] + coff
                db[pl.ds(r * L, L)] = dstbig[sl]

        def scatter_start(j, rb, db, ss, ds_):
            pltpu.async_copy(rb, sh_agg.at[db], ss, add=True)

            @pl.when(deg_on(j))
            def _():
                pltpu.async_copy(ones_v, sh_deg.at[db], ds_, add=True)

        def scatter_wait(j, rb, db, ss, ds_):
            pltpu.make_async_copy(rb, sh_agg.at[db], ss).wait()

            @pl.when(deg_on(j))
            def _():
                pltpu.make_async_copy(ones_v, sh_deg.at[db], ds_).wait()

        def gather_start(ib, rb, gs):
            pltpu.async_copy(x_hbm.at[ib], rb, gs)

        def gather_wait(ib, rb, gs):
            pltpu.make_async_copy(x_hbm.at[ib], rb, gs).wait()

        def halfstep(j, P, PN):
            ib, db, rb, gs, ss, ds_ = P
            ibn, dbn, rbn, gsn, ssn, dsn = PN

            @pl.when(j >= 2)
            def _():
                scatter_wait(j - 2, rbn, dbn, ssn, dsn)

            @pl.when((j + 1) % SUPER == 0)
            def _():
                superload((j + 1) // SUPER)

            prep(j + 1, ibn, dbn)
            gather_wait(ib, rb, gs)
            gather_start(ibn, rbn, gsn)
            scatter_start(j, rb, db, ss, ds_)

        B0 = (idx0, dst0, rows0, gsem0, ssem0, dsem0)
        B1 = (idx1, dst1, rows1, gsem1, ssem1, dsem1)
        B2 = (idx2, dst2, rows2, gsem2, ssem2, dsem2)

        @pl.loop(0, C)
        def _(i):
            for j in range(DH // L):
                rows1[i, pl.ds(j * L, L)] = jnp.zeros((L,), jnp.float32)

        zcopies = []
        off = 0
        for ln in zlens:
            zcopies.append((rows1.at[pl.ds(0, ln)],
                            sh_agg.at[pl.ds(fb + off, ln)], ssem0))
            zcopies.append((zdeg_v.at[pl.ds(0, ln)],
                            sh_deg.at[pl.ds(fb + off, ln)], dsem0))
            off += ln
        for zsrc, zdst, zsm in zcopies:
            pltpu.async_copy(zsrc, zdst, zsm)

        @pl.when(s == NSUB - 1)
        def _():
            tb = NSUB * FLUSH_ROWS
            pltpu.async_copy(rows1.at[pl.ds(0, N - tb)],
                             sh_agg.at[pl.ds(tb, N - tb)], ssem0)
            pltpu.async_copy(zdeg_v.at[pl.ds(0, N - tb)],
                             sh_deg.at[pl.ds(tb, N - tb)], dsem0)

        superload(0)
        prep(0, idx0, dst0)
        gather_start(idx0, rows0, gsem0)

        for zsrc, zdst, zsm in zcopies:
            pltpu.make_async_copy(zsrc, zdst, zsm).wait()

        @pl.when(s == NSUB - 1)
        def _():
            tb = NSUB * FLUSH_ROWS
            pltpu.make_async_copy(rows1.at[pl.ds(0, N - tb)],
                                  sh_agg.at[pl.ds(tb, N - tb)], ssem0).wait()
            pltpu.make_async_copy(zdeg_v.at[pl.ds(0, N - tb)],
                                  sh_deg.at[pl.ds(tb, N - tb)], dsem0).wait()
        plsc.subcore_barrier()

        @pl.loop(0, (CPT - 2) // 3)
        def _(t):
            halfstep(3 * t, B0, B1)
            halfstep(3 * t + 1, B1, B2)
            halfstep(3 * t + 2, B2, B0)

        scatter_wait(CPT - 4, rows1, dst1, ssem1, dsem1)
        prep(CPT - 1, idx1, dst1)
        gather_wait(idx0, rows0, gsem0)
        gather_start(idx1, rows1, gsem1)
        scatter_start(CPT - 2, rows0, dst0, ssem0, dsem0)

        scatter_wait(CPT - 3, rows2, dst2, ssem2, dsem2)
        gather_wait(idx1, rows1, gsem1)
        scatter_start(CPT - 1, rows1, dst1, ssem1, dsem1)

        scatter_wait(CPT - 2, rows0, dst0, ssem0, dsem0)
        scatter_wait(CPT - 1, rows1, dst1, ssem1, dsem1)

        plsc.subcore_barrier()

        fstripes = []
        off = 0
        for ln in zlens:
            fstripes.append((off, ln))
            off += ln
        fbufs = [rows0, rows1, rows2]
        fsems = [ssem0, ssem1, ssem2]
        ns = len(fstripes)

        pltpu.sync_copy(sh_deg.at[pl.ds(fb, FLUSH_ROWS)],
                        dbuf.at[pl.ds(0, FLUSH_ROWS)])
        pltpu.async_copy(dbuf.at[pl.ds(0, FLUSH_ROWS)],
                         deg_hbm.at[pl.ds(coff + fb, FLUSH_ROWS)], dsem0)

        o0, l0 = fstripes[0]
        pltpu.sync_copy(sh_agg.at[pl.ds(fb + o0, l0)],
                        fbufs[0].at[pl.ds(0, l0)])
        for kk in range(ns):
            ok, lk = fstripes[kk]
            pltpu.async_copy(fbufs[kk % 3].at[pl.ds(0, lk)],
                             agg_hbm.at[pl.ds(coff + fb + ok, lk)],
                             fsems[kk % 3])
            if kk + 1 < ns:
                if kk + 1 >= 3:
                    op, lp = fstripes[kk + 1 - 3]
                    pltpu.make_async_copy(
                        fbufs[(kk + 1) % 3].at[pl.ds(0, lp)],
                        agg_hbm.at[pl.ds(coff + fb + op, lp)],
                        fsems[(kk + 1) % 3]).wait()
                on, ln_ = fstripes[kk + 1]
                pltpu.sync_copy(sh_agg.at[pl.ds(fb + on, ln_)],
                                fbufs[(kk + 1) % 3].at[pl.ds(0, ln_)])
        for kk in range(max(0, ns - 3), ns):
            ok, lk = fstripes[kk]
            pltpu.make_async_copy(fbufs[kk % 3].at[pl.ds(0, lk)],
                                  agg_hbm.at[pl.ds(coff + fb + ok, lk)],
                                  fsems[kk % 3]).wait()
        pltpu.make_async_copy(dbuf.at[pl.ds(0, FLUSH_ROWS)],
                              deg_hbm.at[pl.ds(coff + fb, FLUSH_ROWS)],
                              dsem0).wait()

        @pl.when(s == NSUB - 1)
        def _():
            tb = NSUB * FLUSH_ROWS
            pltpu.sync_copy(sh_agg.at[pl.ds(tb, N - tb)],
                            rows0.at[pl.ds(0, N - tb)])
            pltpu.sync_copy(rows0.at[pl.ds(0, N - tb)],
                            agg_hbm.at[pl.ds(coff + tb, N - tb)])
            pltpu.sync_copy(sh_deg.at[pl.ds(tb, N - tb)],
                            dbuf.at[pl.ds(0, N - tb)])
            pltpu.sync_copy(dbuf.at[pl.ds(0, N - tb)],
                            deg_hbm.at[pl.ds(coff + tb, N - tb)])

    return k(x_cat, src, dst)


def _norm_body(xu, aul, auh, dul, duh, xi, ail, aih, dil, dih,
               pu, pi, gu, bu, gi, bi, ou, oi):
    for x, alo, ahi, dlo, dhi, p, g, b, o in (
            (xu, aul, auh, dul, duh, pu, gu, bu, ou),
            (xi, ail, aih, dil, dih, pi, gi, bi, oi)):
        a = jnp.concatenate([alo[...], ahi[...]], axis=1)
        d = jnp.maximum(dlo[...] + dhi[...], 1.0)
        h = x[...] + (ALPHA * (a * p[...])) / d
        mu = jnp.mean(h, axis=-1, keepdims=True)
        var = jnp.mean((h - mu) ** 2, axis=-1, keepdims=True)
        o[...] = (h - mu) * lax.rsqrt(var + EPS) * g[...] + b[...]


def _tc_normalize(x_user, aggu_cat, degu_cat, x_item, aggi_cat, degi_cat,
                  p_iu, p_ui, g_u, b_u, g_i, b_i):
    R = 1000
    bs_feat = pl.BlockSpec((R, D), lambda i: (i, 0))
    bs_lo = pl.BlockSpec((R, DH), lambda i: (i, 0))
    bs_hi = pl.BlockSpec((R, DH), lambda i: (N // R + i, 0))
    bs_dlo = pl.BlockSpec((R, 1), lambda i: (i, 0))
    bs_dhi = pl.BlockSpec((R, 1), lambda i: (N // R + i, 0))
    bs_vec = pl.BlockSpec((1, D), lambda i: (0, 0))
    return pl.pallas_call(
        _norm_body,
        grid=(N // R,),
        in_specs=[bs_feat, bs_lo, bs_hi, bs_dlo, bs_dhi,
                  bs_feat, bs_lo, bs_hi, bs_dlo, bs_dhi,
                  bs_vec, bs_vec, bs_vec, bs_vec, bs_vec, bs_vec],
        out_specs=[bs_feat, bs_feat],
        out_shape=[jax.ShapeDtypeStruct((N, D), jnp.float32),
                   jax.ShapeDtypeStruct((N, D), jnp.float32)],
    )(x_user, aggu_cat, aggu_cat, degu_cat, degu_cat,
      x_item, aggi_cat, aggi_cat, degi_cat, degi_cat,
      p_iu.reshape(1, D), p_ui.reshape(1, D),
      g_u.reshape(1, D), b_u.reshape(1, D),
      g_i.reshape(1, D), b_i.reshape(1, D))


def kernel(x_user, x_item, edge_user_item, edge_item_user,
           p_user_item, p_item_user, g_user, b_user, g_item, b_item):
    xu_cat = jnp.concatenate([x_user[:, :DH], x_user[:, DH:]], axis=0)
    xi_cat = jnp.concatenate([x_item[:, :DH], x_item[:, DH:]], axis=0)
    aggi_cat, degi_cat = _sc_aggregate(
        xu_cat, edge_user_item[0], edge_user_item[1])
    aggu_cat, degu_cat = _sc_aggregate(
        xi_cat, edge_item_user[0], edge_item_user[1])
    out_user, out_item = _tc_normalize(
        x_user, aggu_cat, degu_cat.reshape(2 * N, 1),
        x_item, aggi_cat, degi_cat.reshape(2 * N, 1),
        p_item_user, p_user_item, g_user, b_user, g_item, b_item)
    return (out_user, out_item)


# speedup vs baseline: 1.0470x; 1.0182x over previous
import functools

import jax
import jax.numpy as jnp
from jax import lax
from jax.experimental import pallas as pl
from jax.experimental.pallas import tpu as pltpu
from jax.experimental.pallas import tpu_sc as plsc

N = 10000
D = 256
E = 160000
DH = 128
C = 80
L = 16
NSUB = 16
EPT = E // NSUB
CPT = EPT // C
DEG_SPLIT = (CPT + 1) // 2
SUPER = 25
SUPER_E = SUPER * C
FLUSH_ROWS = (N // NSUB) // 8 * 8
ALPHA = 0.5
EPS = 1e-5


def _sc_aggregate(x_cat, src, dst):
    mesh = plsc.VectorSubcoreMesh(core_axis_name="core",
                                  subcore_axis_name="subcore")
    out_type = [
        jax.ShapeDtypeStruct((2 * N, DH), jnp.float32),
        jax.ShapeDtypeStruct((N,), jnp.float32),
    ]
    scratch = [
        pltpu.VMEM_SHARED((N, DH), jnp.float32),
        pltpu.VMEM_SHARED((N,), jnp.float32),
        pltpu.VMEM((SUPER_E,), jnp.int32),
        pltpu.VMEM((SUPER_E,), jnp.int32),
        pltpu.VMEM((C,), jnp.int32),
        pltpu.VMEM((C,), jnp.int32),
        pltpu.VMEM((C, DH), jnp.float32),
        pltpu.VMEM((C,), jnp.int32),
        pltpu.VMEM((C,), jnp.int32),
        pltpu.VMEM((C, DH), jnp.float32),
        pltpu.VMEM((C,), jnp.int32),
        pltpu.VMEM((C,), jnp.int32),
        pltpu.VMEM((C, DH), jnp.float32),
        pltpu.VMEM((C,), jnp.float32),
        pltpu.VMEM((C,), jnp.float32),
        pltpu.VMEM((FLUSH_ROWS + 16,), jnp.float32),
        pltpu.SemaphoreType.DMA,
        pltpu.SemaphoreType.DMA,
        pltpu.SemaphoreType.DMA,
        pltpu.SemaphoreType.DMA,
        pltpu.SemaphoreType.DMA,
        pltpu.SemaphoreType.DMA,
        pltpu.SemaphoreType.DMA,
        pltpu.SemaphoreType.DMA,
        pltpu.SemaphoreType.DMA,
    ]

    @functools.partial(pl.kernel, mesh=mesh, out_type=out_type,
                       scratch_types=scratch)
    def k(x_hbm, s_hbm, d_hbm, agg_hbm, deg_hbm,
          sh_agg, sh_deg, srcbig, dstbig,
          idx0, dst0, rows0, idx1, dst1, rows1, idx2, dst2, rows2,
          ones_v, zdeg_v, dbuf,
          gsem0, gsem1, gsem2, ssem0, ssem1, ssem2, dsem0, dsem1, dsem2):
        c = lax.axis_index("core")
        s = lax.axis_index("subcore")
        coff = c * N
        ebase = s * EPT

        for j in range(C // L):
            ones_v[pl.ds(j * L, L)] = jnp.full((L,), 1.0, jnp.float32)
            zdeg_v[pl.ds(j * L, L)] = jnp.zeros((L,), jnp.float32)

        fb = s * FLUSH_ROWS
        zlens = [C] * (FLUSH_ROWS // C) + [FLUSH_ROWS % C]

        def deg_on(j):
            del j
            return c == 0

        def superload(g):
            e0 = ebase + g * SUPER_E
            pltpu.sync_copy(s_hbm.at[pl.ds(e0, SUPER_E)], srcbig)
            pltpu.sync_copy(d_hbm.at[pl.ds(e0, SUPER_E)], dstbig)

        def prep(j, ib, db):
            off_ = (j % SUPER) * C
            for r in range(C // L):
                sl = pl.ds(off_ + r * L, L)
                ib[pl.ds(r * L, L)] = srcbig[sl] + coff
                db[pl.ds(r * L, L)] = dstbig[sl]

        def scatter_start(j, rb, db, ss, ds_):
            pltpu.async_copy(rb, sh_agg.at[db], ss, add=True)

            @pl.when(deg_on(j))
            def _():
                pltpu.async_copy(ones_v, sh_deg.at[db], ds_, add=True)

        def scatter_wait(j, rb, db, ss, ds_):
            pltpu.make_async_copy(rb, sh_agg.at[db], ss).wait()

            @pl.when(deg_on(j))
            def _():
                pltpu.make_async_copy(ones_v, sh_deg.at[db], ds_).wait()

        def gather_start(ib, rb, gs):
            pltpu.async_copy(x_hbm.at[ib], rb, gs)

        def gather_wait(ib, rb, gs):
            pltpu.make_async_copy(x_hbm.at[ib], rb, gs).wait()

        def halfstep(j, P, PN):
            ib, db, rb, gs, ss, ds_ = P
            ibn, dbn, rbn, gsn, ssn, dsn = PN

            @pl.when(j >= 2)
            def _():
                scatter_wait(j - 2, rbn, dbn, ssn, dsn)

            @pl.when((j + 1) % SUPER == 0)
            def _():
                superload((j + 1) // SUPER)

            prep(j + 1, ibn, dbn)
            gather_wait(ib, rb, gs)
            gather_start(ibn, rbn, gsn)
            scatter_start(j, rb, db, ss, ds_)

        B0 = (idx0, dst0, rows0, gsem0, ssem0, dsem0)
        B1 = (idx1, dst1, rows1, gsem1, ssem1, dsem1)
        B2 = (idx2, dst2, rows2, gsem2, ssem2, dsem2)

        @pl.loop(0, C)
        def _(i):
            for j in range(DH // L):
                rows1[i, pl.ds(j * L, L)] = jnp.zeros((L,), jnp.float32)

        zcopies = []
        off = 0
        for ln in zlens:
            zcopies.append((rows1.at[pl.ds(0, ln)],
                            sh_agg.at[pl.ds(fb + off, ln)], ssem0))
            zcopies.append((zdeg_v.at[pl.ds(0, ln)],
                            sh_deg.at[pl.ds(fb + off, ln)], dsem0))
            off += ln
        for zsrc, zdst, zsm in zcopies:
            pltpu.async_copy(zsrc, zdst, zsm)

        @pl.when(s == NSUB - 1)
        def _():
            tb = NSUB * FLUSH_ROWS
            pltpu.async_copy(rows1.at[pl.ds(0, N - tb)],
                             sh_agg.at[pl.ds(tb, N - tb)], ssem0)
            pltpu.async_copy(zdeg_v.at[pl.ds(0, N - tb)],
                             sh_deg.at[pl.ds(tb, N - tb)], dsem0)

        superload(0)
        prep(0, idx0, dst0)
        gather_start(idx0, rows0, gsem0)

        for zsrc, zdst, zsm in zcopies:
            pltpu.make_async_copy(zsrc, zdst, zsm).wait()

        @pl.when(s == NSUB - 1)
        def _():
            tb = NSUB * FLUSH_ROWS
            pltpu.make_async_copy(rows1.at[pl.ds(0, N - tb)],
                                  sh_agg.at[pl.ds(tb, N - tb)], ssem0).wait()
            pltpu.make_async_copy(zdeg_v.at[pl.ds(0, N - tb)],
                                  sh_deg.at[pl.ds(tb, N - tb)], dsem0).wait()
        plsc.subcore_barrier()

        @pl.loop(0, (CPT - 2) // 3)
        def _(t):
            halfstep(3 * t, B0, B1)
            halfstep(3 * t + 1, B1, B2)
            halfstep(3 * t + 2, B2, B0)

        scatter_wait(CPT - 4, rows1, dst1, ssem1, dsem1)
        prep(CPT - 1, idx1, dst1)
        gather_wait(idx0, rows0, gsem0)
        gather_start(idx1, rows1, gsem1)
        scatter_start(CPT - 2, rows0, dst0, ssem0, dsem0)

        scatter_wait(CPT - 3, rows2, dst2, ssem2, dsem2)
        gather_wait(idx1, rows1, gsem1)
        scatter_start(CPT - 1, rows1, dst1, ssem1, dsem1)

        scatter_wait(CPT - 2, rows0, dst0, ssem0, dsem0)
        scatter_wait(CPT - 1, rows1, dst1, ssem1, dsem1)

        plsc.subcore_barrier()

        fstripes = []
        off = 0
        for ln in zlens:
            fstripes.append((off, ln))
            off += ln
        fbufs = [rows0, rows1, rows2]
        fsems = [ssem0, ssem1, ssem2]
        ns = len(fstripes)

        @pl.when(c == 0)
        def _():
            pltpu.sync_copy(sh_deg.at[pl.ds(fb, FLUSH_ROWS)],
                            dbuf.at[pl.ds(0, FLUSH_ROWS)])
            pltpu.async_copy(dbuf.at[pl.ds(0, FLUSH_ROWS)],
                             deg_hbm.at[pl.ds(fb, FLUSH_ROWS)], dsem0)

        o0, l0 = fstripes[0]
        pltpu.sync_copy(sh_agg.at[pl.ds(fb + o0, l0)],
                        fbufs[0].at[pl.ds(0, l0)])
        for kk in range(ns):
            ok, lk = fstripes[kk]
            pltpu.async_copy(fbufs[kk % 3].at[pl.ds(0, lk)],
                             agg_hbm.at[pl.ds(coff + fb + ok, lk)],
                             fsems[kk % 3])
            if kk + 1 < ns:
                if kk + 1 >= 3:
                    op, lp = fstripes[kk + 1 - 3]
                    pltpu.make_async_copy(
                        fbufs[(kk + 1) % 3].at[pl.ds(0, lp)],
                        agg_hbm.at[pl.ds(coff + fb + op, lp)],
                        fsems[(kk + 1) % 3]).wait()
                on, ln_ = fstripes[kk + 1]
                pltpu.sync_copy(sh_agg.at[pl.ds(fb + on, ln_)],
                                fbufs[(kk + 1) % 3].at[pl.ds(0, ln_)])
        for kk in range(max(0, ns - 3), ns):
            ok, lk = fstripes[kk]
            pltpu.make_async_copy(fbufs[kk % 3].at[pl.ds(0, lk)],
                                  agg_hbm.at[pl.ds(coff + fb + ok, lk)],
                                  fsems[kk % 3]).wait()
        @pl.when(c == 0)
        def _():
            pltpu.make_async_copy(dbuf.at[pl.ds(0, FLUSH_ROWS)],
                                  deg_hbm.at[pl.ds(fb, FLUSH_ROWS)],
                                  dsem0).wait()

        @pl.when(s == NSUB - 1)
        def _():
            tb = NSUB * FLUSH_ROWS
            pltpu.sync_copy(sh_agg.at[pl.ds(tb, N - tb)],
                            rows0.at[pl.ds(0, N - tb)])
            pltpu.sync_copy(rows0.at[pl.ds(0, N - tb)],
                            agg_hbm.at[pl.ds(coff + tb, N - tb)])
            @pl.when(c == 0)
            def _():
                pltpu.sync_copy(sh_deg.at[pl.ds(tb, N - tb)],
                                dbuf.at[pl.ds(0, N - tb)])
                pltpu.sync_copy(dbuf.at[pl.ds(0, N - tb)],
                                deg_hbm.at[pl.ds(tb, N - tb)])

    return k(x_cat, src, dst)


def _norm_body(xu, aul, auh, du, xi, ail, aih, di,
               pu, pi, gu, bu, gi, bi, ou, oi):
    for x, alo, ahi, d2, p, g, b, o in (
            (xu, aul, auh, du, pu, gu, bu, ou),
            (xi, ail, aih, di, pi, gi, bi, oi)):
        a = jnp.concatenate([alo[...], ahi[...]], axis=1)
        d = jnp.maximum(d2[...], 1.0)
        h = x[...] + (ALPHA * (a * p[...])) / d
        mu = jnp.mean(h, axis=-1, keepdims=True)
        var = jnp.mean((h - mu) ** 2, axis=-1, keepdims=True)
        o[...] = (h - mu) * lax.rsqrt(var + EPS) * g[...] + b[...]


def _tc_normalize(x_user, aggu_cat, degu_cat, x_item, aggi_cat, degi_cat,
                  p_iu, p_ui, g_u, b_u, g_i, b_i):
    R = 1000
    bs_feat = pl.BlockSpec((R, D), lambda i: (i, 0))
    bs_lo = pl.BlockSpec((R, DH), lambda i: (i, 0))
    bs_hi = pl.BlockSpec((R, DH), lambda i: (N // R + i, 0))
    bs_deg = pl.BlockSpec((R, 1), lambda i: (i, 0))
    bs_vec = pl.BlockSpec((1, D), lambda i: (0, 0))
    return pl.pallas_call(
        _norm_body,
        grid=(N // R,),
        in_specs=[bs_feat, bs_lo, bs_hi, bs_deg,
                  bs_feat, bs_lo, bs_hi, bs_deg,
                  bs_vec, bs_vec, bs_vec, bs_vec, bs_vec, bs_vec],
        out_specs=[bs_feat, bs_feat],
        out_shape=[jax.ShapeDtypeStruct((N, D), jnp.float32),
                   jax.ShapeDtypeStruct((N, D), jnp.float32)],
    )(x_user, aggu_cat, aggu_cat, degu_cat,
      x_item, aggi_cat, aggi_cat, degi_cat,
      p_iu.reshape(1, D), p_ui.reshape(1, D),
      g_u.reshape(1, D), b_u.reshape(1, D),
      g_i.reshape(1, D), b_i.reshape(1, D))


def kernel(x_user, x_item, edge_user_item, edge_item_user,
           p_user_item, p_item_user, g_user, b_user, g_item, b_item):
    xu_cat = jnp.concatenate([x_user[:, :DH], x_user[:, DH:]], axis=0)
    xi_cat = jnp.concatenate([x_item[:, :DH], x_item[:, DH:]], axis=0)
    aggi_cat, degi_cat = _sc_aggregate(
        xu_cat, edge_user_item[0], edge_user_item[1])
    aggu_cat, degu_cat = _sc_aggregate(
        xi_cat, edge_item_user[0], edge_item_user[1])
    out_user, out_item = _tc_normalize(
        x_user, aggu_cat, degu_cat.reshape(N, 1),
        x_item, aggi_cat, degi_cat.reshape(N, 1),
        p_item_user, p_user_item, g_user, b_user, g_item, b_item)
    return (out_user, out_item)
